# Initial kernel scaffold; baseline (speedup 1.0000x reference)
import jax
import jax.numpy as jnp
from jax.experimental import pallas as pl


def kernel(x, edge_index, W1, b1, W2, b2):
    raise NotImplementedError("write your pallas kernel here")


# speedup vs baseline: 5.1557x; 5.1557x over previous
import functools

import jax
import jax.numpy as jnp
from jax import lax
from jax.experimental import pallas as pl
from jax.experimental.pallas import tpu as pltpu
from jax.experimental.pallas import tpu_sc as plsc

NC = 2
NS = 16
NW = NC * NS
L = 16


def _sc_agg_body(npt, chunks_per_worker, k, d,
                 x_hbm, src_hbm, dst_hbm, agg_out,
                 sidx, didx, xrows, zagg, shared_agg, sem):
    c = lax.axis_index("c")
    s = lax.axis_index("s")
    w = c * NS + s
    epw = chunks_per_worker * k
    zrows = zagg.shape[0]

    def fz(i, _):
        for hh in range(d // L):
            zagg[i, pl.ds(hh * L, L)] = jnp.zeros((L,), jnp.float32)
        return 0
    lax.fori_loop(0, zrows, fz, 0)

    for j in range(npt // zrows):
        pltpu.sync_copy(zagg, shared_agg.at[pl.ds(s * npt + j * zrows, zrows)])
    plsc.subcore_barrier()

    pltpu.sync_copy(src_hbm.at[pl.ds(w * epw, epw)], sidx)
    pltpu.sync_copy(dst_hbm.at[w], didx)

    def chunk(t, _):
        pltpu.async_copy(x_hbm.at[sidx.at[pl.ds(t * k, k)]], xrows, sem).wait()
        pltpu.sync_copy(xrows, shared_agg.at[didx.at[t]], add=True)
        return 0
    lax.fori_loop(0, chunks_per_worker, chunk, 0)

    plsc.subcore_barrier()
    pltpu.sync_copy(shared_agg.at[pl.ds(s * npt, npt)], agg_out.at[w])


def _sc_deg_body(npt, chunks_per_worker, k, d,
                 dst_hbm, deg_out, didx, ones_v, zdeg, shared_deg, sem):
    c = lax.axis_index("c")
    s = lax.axis_index("s")
    w = c * NS + s
    zrows = zdeg.shape[0]

    def fz(i, _):
        for hh in range(d // L):
            zdeg[i, pl.ds(hh * L, L)] = jnp.zeros((L,), jnp.float32)
        return 0
    lax.fori_loop(0, zrows, fz, 0)

    def fo(i, _):
        for hh in range(d // L):
            ones_v[i, pl.ds(hh * L, L)] = jnp.ones((L,), jnp.float32)
        return 0
    lax.fori_loop(0, k, fo, 0)

    for j in range(npt // zrows):
        pltpu.sync_copy(zdeg, shared_deg.at[pl.ds(s * npt + j * zrows, zrows)])
    plsc.subcore_barrier()

    pltpu.sync_copy(dst_hbm.at[w], didx)

    def chunk(t, _):
        pltpu.sync_copy(ones_v, shared_deg.at[didx.at[t]], add=True)
        return 0
    lax.fori_loop(0, chunks_per_worker, chunk, 0)

    plsc.subcore_barrier()
    pltpu.sync_copy(shared_deg.at[pl.ds(s * npt, npt)], deg_out.at[w])


def _sc_edge_body(chunks_per_worker, k, d,
                  nl_hbm, src_hbm, dst_hbm, out_hbm,
                  sidx, didx, abuf, bbuf, sem):
    c = lax.axis_index("c")
    s = lax.axis_index("s")
    w = c * NS + s

    edges_per_worker = chunks_per_worker * k
    pltpu.sync_copy(src_hbm.at[pl.ds(w * edges_per_worker, edges_per_worker)], sidx)
    pltpu.sync_copy(dst_hbm.at[pl.ds(w * edges_per_worker, edges_per_worker)], didx)

    def chunk(t, _):
        pltpu.async_copy(nl_hbm.at[sidx.at[pl.ds(t * k, k)]], abuf, sem).wait()
        pltpu.async_copy(nl_hbm.at[didx.at[pl.ds(t * k, k)]], bbuf, sem).wait()

        def row(i, _):
            for hh in range(d // L):
                sl = pl.ds(hh * L, L)
                plsc.addupdate(abuf.at[i, sl], bbuf[i, sl])
            return 0
        lax.fori_loop(0, k, row, 0)

        pltpu.sync_copy(abuf, out_hbm.at[pl.ds(w * edges_per_worker + t * k, k)])
        return 0
    lax.fori_loop(0, chunks_per_worker, chunk, 0)


def _tc_fuse_body(x_ref, agg_ref, deg_ref, w1_ref, b1_ref, w2_ref, b2_ref, o_ref):
    xb = x_ref[...]
    a = agg_ref[0] + agg_ref[1]
    dg = deg_ref[0, :, 0:1] + deg_ref[1, :, 0:1]
    invd = 1.0 / jnp.maximum(dg, 1.0)
    ind = jnp.minimum(dg, 1.0)
    pre = jnp.dot(xb + a * invd, w1_ref[...],
                  preferred_element_type=jnp.float32)
    pre = pre + b1_ref[...] * (1.0 + ind)
    hid = jnp.maximum(pre, 0.0)
    o_ref[...] = (jnp.dot(hid, w2_ref[...],
                          preferred_element_type=jnp.float32) * 0.5
                  + b2_ref[...] * 0.5)


@functools.partial(jax.jit, static_argnames=("interpret",))
def _run(x, edge_index, W1, b1, W2, b2, interpret=False):
    n, d = x.shape
    e = edge_index.shape[1]
    h = W1.shape[1]
    out_d = W2.shape[1]
    k = 80
    assert e % (NW * k) == 0 and d % L == 0
    cpw = e // (NW * k)
    npt = ((n + NS * 128 - 1) // (NS * 128)) * 128
    n_pad = npt * NS

    src_flat = edge_index[0]
    dst_flat = edge_index[1]
    dst3d = dst_flat.reshape(NW, cpw, k)
    epw = cpw * k

    mesh = plsc.VectorSubcoreMesh(core_axis_name="c", subcore_axis_name="s",
                                  num_cores=NC, num_subcores=NS)

    agg_t = pl.kernel(
        functools.partial(_sc_agg_body, npt, cpw, k, d),
        out_type=jax.ShapeDtypeStruct((NW, npt, d), jnp.float32),
        mesh=mesh,
        scratch_types=[
            pltpu.VMEM((epw,), jnp.int32),
            pltpu.VMEM((cpw, k), jnp.int32),
            pltpu.VMEM((k, d), jnp.float32),
            pltpu.VMEM((40, d), jnp.float32),
            pltpu.VMEM_SHARED((n_pad, d), jnp.float32),
            pltpu.SemaphoreType.DMA,
        ],
        interpret=interpret,
    )(x, src_flat, dst3d)

    deg_t = pl.kernel(
        functools.partial(_sc_deg_body, npt, cpw, k, d),
        out_type=jax.ShapeDtypeStruct((NW, npt, d), jnp.float32),
        mesh=mesh,
        scratch_types=[
            pltpu.VMEM((cpw, k), jnp.int32),
            pltpu.VMEM((k, d), jnp.float32),
            pltpu.VMEM((40, d), jnp.float32),
            pltpu.VMEM_SHARED((n_pad, d), jnp.float32),
            pltpu.SemaphoreType.DMA,
        ],
        interpret=interpret,
    )(dst3d)

    agg = agg_t.reshape(NC, n_pad, d)
    deg = deg_t.reshape(NC, n_pad, d)

    bn = 1000
    grid = (n // bn,)
    nl_half = pl.pallas_call(
        _tc_fuse_body,
        grid=grid,
        in_specs=[
            pl.BlockSpec((bn, d), lambda i: (i, 0)),
            pl.BlockSpec((NC, bn, d), lambda i: (0, i, 0)),
            pl.BlockSpec((NC, bn, d), lambda i: (0, i, 0)),
            pl.BlockSpec((d, h), lambda i: (0, 0)),
            pl.BlockSpec((1, h), lambda i: (0, 0)),
            pl.BlockSpec((h, out_d), lambda i: (0, 0)),
            pl.BlockSpec((1, out_d), lambda i: (0, 0)),
        ],
        out_specs=pl.BlockSpec((bn, out_d), lambda i: (i, 0)),
        out_shape=jax.ShapeDtypeStruct((n, out_d), jnp.float32),
        interpret=interpret,
    )(x, agg, deg, W1, b1.reshape(1, h), W2, b2.reshape(1, out_d))

    edge_logits = pl.kernel(
        functools.partial(_sc_edge_body, cpw, k, out_d),
        out_type=jax.ShapeDtypeStruct((e, out_d), jnp.float32),
        mesh=mesh,
        scratch_types=[
            pltpu.VMEM((epw,), jnp.int32),
            pltpu.VMEM((epw,), jnp.int32),
            pltpu.VMEM((k, out_d), jnp.float32),
            pltpu.VMEM((k, out_d), jnp.float32),
            pltpu.SemaphoreType.DMA,
        ],
        interpret=interpret,
    )(nl_half, src_flat, dst_flat)

    return edge_logits


def kernel(x, edge_index, W1, b1, W2, b2):
    return _run(x, edge_index, W1, b1, W2, b2)


# speedup vs baseline: 8.7676x; 1.7006x over previous
import functools

import jax
import jax.numpy as jnp
from jax import lax
from jax.experimental import pallas as pl
from jax.experimental.pallas import tpu as pltpu
from jax.experimental.pallas import tpu_sc as plsc

NC = 2
NS = 16
NW = NC * NS
L = 16


def _sc_agg_body(npt, chunks_per_worker, k, d,
                 x_hbm, src_hbm, dst_hbm, zeros_hbm, agg_out,
                 sidx, didx, xrows0, xrows1, shared_agg,
                 gsem0, gsem1, asem0, asem1):
    c = lax.axis_index("c")
    s = lax.axis_index("s")
    w = c * NS + s
    epw = chunks_per_worker * k

    pltpu.sync_copy(zeros_hbm.at[pl.ds(s * npt, npt)],
                    shared_agg.at[pl.ds(s * npt, npt)])
    plsc.subcore_barrier()

    pltpu.sync_copy(src_hbm.at[pl.ds(w * epw, epw)], sidx)
    pltpu.sync_copy(dst_hbm.at[w], didx)

    xr = (xrows0, xrows1)
    gs = (gsem0, gsem1)
    asems = (asem0, asem1)
    cpw = chunks_per_worker

    pltpu.async_copy(x_hbm.at[sidx.at[pl.ds(0, k)]], xr[0], gs[0])

    def chunk(t, _):
        for p in (0, 1):
            q = 1 - p

            @pl.when(t % 2 == p)
            def _():
                @pl.when(jnp.logical_and(t + 1 < cpw, t >= 1))
                def _():
                    pltpu.make_async_copy(
                        xr[q], shared_agg.at[didx.at[t]], asems[q]).wait()

                @pl.when(t + 1 < cpw)
                def _():
                    pltpu.async_copy(
                        x_hbm.at[sidx.at[pl.ds((t + 1) * k, k)]], xr[q], gs[q])

                pltpu.make_async_copy(
                    x_hbm.at[sidx.at[pl.ds(t * k, k)]], xr[p], gs[p]).wait()
                pltpu.async_copy(
                    xr[p], shared_agg.at[didx.at[t]], asems[p], add=True)
        return 0
    lax.fori_loop(0, cpw, chunk, 0)

    last = (cpw - 1) % 2
    pltpu.make_async_copy(xr[last], shared_agg.at[didx.at[0]], asems[last]).wait()
    pltpu.make_async_copy(xr[1 - last], shared_agg.at[didx.at[0]], asems[1 - last]).wait()
    plsc.subcore_barrier()
    pltpu.sync_copy(shared_agg.at[pl.ds(s * npt, npt)], agg_out.at[w])


def _sc_deg_body(npt, chunks_per_worker, k, d,
                 dst_hbm, zeros_hbm, ones_hbm, deg_out,
                 didx, ones_v, shared_deg, sem):
    c = lax.axis_index("c")
    s = lax.axis_index("s")
    w = c * NS + s

    pltpu.sync_copy(zeros_hbm.at[pl.ds(s * npt, npt)],
                    shared_deg.at[pl.ds(s * npt, npt)])
    pltpu.sync_copy(ones_hbm, ones_v)
    plsc.subcore_barrier()

    pltpu.sync_copy(dst_hbm.at[w], didx)

    def chunk(t, _):
        pltpu.sync_copy(ones_v, shared_deg.at[didx.at[t]], add=True)
        return 0
    lax.fori_loop(0, chunks_per_worker, chunk, 0)

    plsc.subcore_barrier()
    pltpu.sync_copy(shared_deg.at[pl.ds(s * npt, npt)], deg_out.at[w])


def _sc_edge_body(chunks_per_worker, k, d,
                  nl_hbm, src_hbm, dst_hbm, out_hbm,
                  sidx, didx, abuf0, abuf1, bbuf0, bbuf1,
                  gsem0, gsem1, ssem0, ssem1):
    c = lax.axis_index("c")
    s = lax.axis_index("s")
    w = c * NS + s

    edges_per_worker = chunks_per_worker * k
    pltpu.sync_copy(src_hbm.at[pl.ds(w * edges_per_worker, edges_per_worker)], sidx)
    pltpu.sync_copy(dst_hbm.at[pl.ds(w * edges_per_worker, edges_per_worker)], didx)

    ab = (abuf0, abuf1)
    bb = (bbuf0, bbuf1)
    gs = (gsem0, gsem1)
    ss = (ssem0, ssem1)
    cpw = chunks_per_worker
    base = w * edges_per_worker

    def start_gathers(t, p):
        pltpu.async_copy(nl_hbm.at[sidx.at[pl.ds(t * k, k)]], ab[p], gs[p])
        pltpu.async_copy(nl_hbm.at[didx.at[pl.ds(t * k, k)]], bb[p], gs[p])

    start_gathers(0, 0)

    def chunk(t, _):
        for p in (0, 1):
            q = 1 - p

            @pl.when(t % 2 == p)
            def _():
                @pl.when(jnp.logical_and(t + 1 < cpw, t >= 1))
                def _():
                    pltpu.make_async_copy(
                        ab[q], out_hbm.at[pl.ds(base + t * k, k)], ss[q]).wait()

                @pl.when(t + 1 < cpw)
                def _():
                    start_gathers(t + 1, q)

                pltpu.make_async_copy(
                    nl_hbm.at[sidx.at[pl.ds(t * k, k)]], ab[p], gs[p]).wait()
                pltpu.make_async_copy(
                    nl_hbm.at[didx.at[pl.ds(t * k, k)]], bb[p], gs[p]).wait()

                def row(i, _):
                    for hh in range(d // L):
                        sl = pl.ds(hh * L, L)
                        plsc.addupdate(ab[p].at[i, sl], bb[p][i, sl])
                    return 0
                lax.fori_loop(0, k, row, 0)

                pltpu.async_copy(
                    ab[p], out_hbm.at[pl.ds(base + t * k, k)], ss[p])
        return 0
    lax.fori_loop(0, cpw, chunk, 0)

    last = (cpw - 1) % 2
    pltpu.make_async_copy(ab[last], out_hbm.at[pl.ds(base, k)], ss[last]).wait()
    pltpu.make_async_copy(ab[1 - last], out_hbm.at[pl.ds(base, k)], ss[1 - last]).wait()


def _tc_fuse_body(x_ref, agg_ref, deg_ref, w1_ref, b1_ref, w2_ref, b2_ref, o_ref):
    xb = x_ref[...]
    a = agg_ref[0] + agg_ref[1]
    dg = deg_ref[0, :, 0:1] + deg_ref[1, :, 0:1]
    invd = 1.0 / jnp.maximum(dg, 1.0)
    ind = jnp.minimum(dg, 1.0)
    pre = jnp.dot(xb + a * invd, w1_ref[...],
                  preferred_element_type=jnp.float32)
    pre = pre + b1_ref[...] * (1.0 + ind)
    hid = jnp.maximum(pre, 0.0)
    o_ref[...] = (jnp.dot(hid, w2_ref[...],
                          preferred_element_type=jnp.float32) * 0.5
                  + b2_ref[...] * 0.5)


@functools.partial(jax.jit, static_argnames=("interpret",))
def _run(x, edge_index, W1, b1, W2, b2, interpret=False):
    n, d = x.shape
    e = edge_index.shape[1]
    h = W1.shape[1]
    out_d = W2.shape[1]
    k = 80
    assert e % (NW * k) == 0 and d % L == 0
    cpw = e // (NW * k)
    npt = ((n + NS * 128 - 1) // (NS * 128)) * 128
    n_pad = npt * NS

    src_flat = edge_index[0]
    dst_flat = edge_index[1]
    dst3d = dst_flat.reshape(NW, cpw, k)
    epw = cpw * k

    mesh = plsc.VectorSubcoreMesh(core_axis_name="c", subcore_axis_name="s",
                                  num_cores=NC, num_subcores=NS)

    zeros_pg = jnp.zeros((n_pad, d), jnp.float32)
    ones_pg = jnp.ones((k, d), jnp.float32)

    agg_t = pl.kernel(
        functools.partial(_sc_agg_body, npt, cpw, k, d),
        out_type=jax.ShapeDtypeStruct((NW, npt, d), jnp.float32),
        mesh=mesh,
        scratch_types=[
            pltpu.VMEM((epw,), jnp.int32),
            pltpu.VMEM((cpw, k), jnp.int32),
            pltpu.VMEM((k, d), jnp.float32),
            pltpu.VMEM((k, d), jnp.float32),
            pltpu.VMEM_SHARED((n_pad, d), jnp.float32),
            pltpu.SemaphoreType.DMA,
            pltpu.SemaphoreType.DMA,
            pltpu.SemaphoreType.DMA,
            pltpu.SemaphoreType.DMA,
        ],
        interpret=interpret,
    )(x, src_flat, dst3d, zeros_pg)

    deg_t = pl.kernel(
        functools.partial(_sc_deg_body, npt, cpw, k, d),
        out_type=jax.ShapeDtypeStruct((NW, npt, d), jnp.float32),
        mesh=mesh,
        scratch_types=[
            pltpu.VMEM((cpw, k), jnp.int32),
            pltpu.VMEM((k, d), jnp.float32),
            pltpu.VMEM_SHARED((n_pad, d), jnp.float32),
            pltpu.SemaphoreType.DMA,
        ],
        interpret=interpret,
    )(dst3d, zeros_pg, ones_pg)

    agg = agg_t.reshape(NC, n_pad, d)
    deg = deg_t.reshape(NC, n_pad, d)

    bn = 1000
    grid = (n // bn,)
    nl_half = pl.pallas_call(
        _tc_fuse_body,
        grid=grid,
        in_specs=[
            pl.BlockSpec((bn, d), lambda i: (i, 0)),
            pl.BlockSpec((NC, bn, d), lambda i: (0, i, 0)),
            pl.BlockSpec((NC, bn, d), lambda i: (0, i, 0)),
            pl.BlockSpec((d, h), lambda i: (0, 0)),
            pl.BlockSpec((1, h), lambda i: (0, 0)),
            pl.BlockSpec((h, out_d), lambda i: (0, 0)),
            pl.BlockSpec((1, out_d), lambda i: (0, 0)),
        ],
        out_specs=pl.BlockSpec((bn, out_d), lambda i: (i, 0)),
        out_shape=jax.ShapeDtypeStruct((n, out_d), jnp.float32),
        interpret=interpret,
    )(x, agg, deg, W1, b1.reshape(1, h), W2, b2.reshape(1, out_d))

    edge_logits = pl.kernel(
        functools.partial(_sc_edge_body, cpw, k, out_d),
        out_type=jax.ShapeDtypeStruct((e, out_d), jnp.float32),
        mesh=mesh,
        scratch_types=[
            pltpu.VMEM((epw,), jnp.int32),
            pltpu.VMEM((epw,), jnp.int32),
            pltpu.VMEM((k, out_d), jnp.float32),
            pltpu.VMEM((k, out_d), jnp.float32),
            pltpu.VMEM((k, out_d), jnp.float32),
            pltpu.VMEM((k, out_d), jnp.float32),
            pltpu.SemaphoreType.DMA,
            pltpu.SemaphoreType.DMA,
            pltpu.SemaphoreType.DMA,
            pltpu.SemaphoreType.DMA,
        ],
        interpret=interpret,
    )(nl_half, src_flat, dst_flat)

    return edge_logits


def kernel(x, edge_index, W1, b1, W2, b2):
    return _run(x, edge_index, W1, b1, W2, b2)


# speedup vs baseline: 9.2165x; 1.0512x over previous
import functools

import jax
import jax.numpy as jnp
from jax import lax
from jax.experimental import pallas as pl
from jax.experimental.pallas import tpu as pltpu
from jax.experimental.pallas import tpu_sc as plsc

NC = 2
NS = 16
NW = NC * NS
L = 16


def _sc_agg_body(npt, chunks_per_worker, k, d,
                 x_hbm, src_hbm, dst_hbm, zeros_hbm, ones_hbm,
                 agg_out, deg_out,
                 sidx, didx, xrows0, xrows1, shared_agg,
                 gsem0, gsem1, asem0, asem1):
    c = lax.axis_index("c")
    s = lax.axis_index("s")
    w = c * NS + s
    epw = chunks_per_worker * k

    pltpu.sync_copy(zeros_hbm.at[pl.ds(s * npt, npt)],
                    shared_agg.at[pl.ds(s * npt, npt)])
    plsc.subcore_barrier()

    pltpu.sync_copy(src_hbm.at[pl.ds(w * epw, epw)], sidx)
    pltpu.sync_copy(dst_hbm.at[w], didx)

    xr = (xrows0, xrows1)
    gs = (gsem0, gsem1)
    asems = (asem0, asem1)
    cpw = chunks_per_worker

    pltpu.async_copy(x_hbm.at[sidx.at[pl.ds(0, k)]], xr[0], gs[0])

    def chunk(t, _):
        for p in (0, 1):
            q = 1 - p

            @pl.when(t % 2 == p)
            def _():
                @pl.when(jnp.logical_and(t + 1 < cpw, t >= 1))
                def _():
                    pltpu.make_async_copy(
                        xr[q], shared_agg.at[didx.at[t]], asems[q]).wait()

                @pl.when(t + 1 < cpw)
                def _():
                    pltpu.async_copy(
                        x_hbm.at[sidx.at[pl.ds((t + 1) * k, k)]], xr[q], gs[q])

                pltpu.make_async_copy(
                    x_hbm.at[sidx.at[pl.ds(t * k, k)]], xr[p], gs[p]).wait()
                pltpu.async_copy(
                    xr[p], shared_agg.at[didx.at[t]], asems[p], add=True)
        return 0
    lax.fori_loop(0, cpw, chunk, 0)

    last = (cpw - 1) % 2
    pltpu.make_async_copy(xr[last], shared_agg.at[didx.at[0]], asems[last]).wait()
    pltpu.make_async_copy(xr[1 - last], shared_agg.at[didx.at[0]], asems[1 - last]).wait()
    plsc.subcore_barrier()
    pltpu.sync_copy(shared_agg.at[pl.ds(s * npt, npt)], agg_out.at[w])
    plsc.subcore_barrier()

    pltpu.sync_copy(zeros_hbm.at[pl.ds(s * npt, npt)],
                    shared_agg.at[pl.ds(s * npt, npt)])
    pltpu.sync_copy(ones_hbm, xr[0])
    plsc.subcore_barrier()

    def dchunk(t, _):
        for p in (0, 1):
            @pl.when(t % 2 == p)
            def _():
                @pl.when(t >= 2)
                def _():
                    pltpu.make_async_copy(
                        xr[0], shared_agg.at[didx.at[t]], asems[p]).wait()
                pltpu.async_copy(
                    xr[0], shared_agg.at[didx.at[t]], asems[p], add=True)
        return 0
    lax.fori_loop(0, cpw, dchunk, 0)
    pltpu.make_async_copy(xr[0], shared_agg.at[didx.at[0]], asems[0]).wait()
    pltpu.make_async_copy(xr[0], shared_agg.at[didx.at[0]], asems[1]).wait()
    plsc.subcore_barrier()
    pltpu.sync_copy(shared_agg.at[pl.ds(s * npt, npt)], deg_out.at[w])


def _sc_edge_body(chunks_per_worker, k, d,
                  nl_hbm, src_hbm, dst_hbm, out_hbm,
                  sidx, didx, abuf0, abuf1, abuf2, bbuf0, bbuf1, bbuf2,
                  gsem0, gsem1, gsem2, ssem0, ssem1, ssem2):
    c = lax.axis_index("c")
    s = lax.axis_index("s")
    w = c * NS + s

    edges_per_worker = chunks_per_worker * k
    pltpu.sync_copy(src_hbm.at[pl.ds(w * edges_per_worker, edges_per_worker)], sidx)
    pltpu.sync_copy(dst_hbm.at[pl.ds(w * edges_per_worker, edges_per_worker)], didx)

    ab = (abuf0, abuf1, abuf2)
    bb = (bbuf0, bbuf1, bbuf2)
    gs = (gsem0, gsem1, gsem2)
    ss = (ssem0, ssem1, ssem2)
    cpw = chunks_per_worker
    base = w * edges_per_worker

    def start_gathers(t, p):
        pltpu.async_copy(nl_hbm.at[sidx.at[pl.ds(t * k, k)]], ab[p], gs[p])
        pltpu.async_copy(nl_hbm.at[didx.at[pl.ds(t * k, k)]], bb[p], gs[p])

    start_gathers(0, 0)
    start_gathers(1, 1)

    def chunk(t, _):
        for p in (0, 1, 2):
            q = (p + 2) % 3

            @pl.when(t % 3 == p)
            def _():
                @pl.when(t + 2 < cpw)
                def _():
                    @pl.when(t >= 1)
                    def _():
                        pltpu.make_async_copy(
                            ab[q], out_hbm.at[pl.ds(base + t * k, k)],
                            ss[q]).wait()
                    start_gathers(t + 2, q)

                pltpu.make_async_copy(
                    nl_hbm.at[sidx.at[pl.ds(t * k, k)]], ab[p], gs[p]).wait()
                pltpu.make_async_copy(
                    nl_hbm.at[didx.at[pl.ds(t * k, k)]], bb[p], gs[p]).wait()

                def row(i, _):
                    for hh in range(d // L):
                        sl = pl.ds(hh * L, L)
                        plsc.addupdate(ab[p].at[i, sl], bb[p][i, sl])
                    return 0
                lax.fori_loop(0, k, row, 0)

                pltpu.async_copy(
                    ab[p], out_hbm.at[pl.ds(base + t * k, k)], ss[p])
        return 0
    lax.fori_loop(0, cpw, chunk, 0)

    for p in (0, 1, 2):
        pltpu.make_async_copy(ab[p], out_hbm.at[pl.ds(base, k)], ss[p]).wait()


def _tc_fuse_body(x_ref, agg_ref, deg_ref, w1_ref, b1_ref, w2_ref, b2_ref, o_ref):
    xb = x_ref[...]
    a = agg_ref[0] + agg_ref[1]
    dg = deg_ref[0, :, 0:1] + deg_ref[1, :, 0:1]
    invd = 1.0 / jnp.maximum(dg, 1.0)
    ind = jnp.minimum(dg, 1.0)
    pre = jnp.dot(xb + a * invd, w1_ref[...],
                  preferred_element_type=jnp.float32)
    pre = pre + b1_ref[...] * (1.0 + ind)
    hid = jnp.maximum(pre, 0.0)
    o_ref[...] = (jnp.dot(hid, w2_ref[...],
                          preferred_element_type=jnp.float32) * 0.5
                  + b2_ref[...] * 0.5)


@functools.partial(jax.jit, static_argnames=("interpret",))
def _run(x, edge_index, W1, b1, W2, b2, interpret=False):
    n, d = x.shape
    e = edge_index.shape[1]
    h = W1.shape[1]
    out_d = W2.shape[1]
    k = 80
    assert e % (NW * k) == 0 and d % L == 0
    cpw = e // (NW * k)
    npt = ((n + NS * 128 - 1) // (NS * 128)) * 128
    n_pad = npt * NS

    src_flat = edge_index[0]
    dst_flat = edge_index[1]
    dst3d = dst_flat.reshape(NW, cpw, k)
    epw = cpw * k

    mesh = plsc.VectorSubcoreMesh(core_axis_name="c", subcore_axis_name="s",
                                  num_cores=NC, num_subcores=NS)

    zeros_pg = jnp.zeros((n_pad, d), jnp.float32)
    ones_pg = jnp.ones((k, d), jnp.float32)

    agg_t, deg_t = pl.kernel(
        functools.partial(_sc_agg_body, npt, cpw, k, d),
        out_type=[
            jax.ShapeDtypeStruct((NW, npt, d), jnp.float32),
            jax.ShapeDtypeStruct((NW, npt, d), jnp.float32),
        ],
        mesh=mesh,
        scratch_types=[
            pltpu.VMEM((epw,), jnp.int32),
            pltpu.VMEM((cpw, k), jnp.int32),
            pltpu.VMEM((k, d), jnp.float32),
            pltpu.VMEM((k, d), jnp.float32),
            pltpu.VMEM_SHARED((n_pad, d), jnp.float32),
            pltpu.SemaphoreType.DMA,
            pltpu.SemaphoreType.DMA,
            pltpu.SemaphoreType.DMA,
            pltpu.SemaphoreType.DMA,
        ],
        interpret=interpret,
    )(x, src_flat, dst3d, zeros_pg, ones_pg)

    agg = agg_t.reshape(NC, n_pad, d)
    deg = deg_t.reshape(NC, n_pad, d)

    bn = 1000
    grid = (n // bn,)
    nl_half = pl.pallas_call(
        _tc_fuse_body,
        grid=grid,
        in_specs=[
            pl.BlockSpec((bn, d), lambda i: (i, 0)),
            pl.BlockSpec((NC, bn, d), lambda i: (0, i, 0)),
            pl.BlockSpec((NC, bn, d), lambda i: (0, i, 0)),
            pl.BlockSpec((d, h), lambda i: (0, 0)),
            pl.BlockSpec((1, h), lambda i: (0, 0)),
            pl.BlockSpec((h, out_d), lambda i: (0, 0)),
            pl.BlockSpec((1, out_d), lambda i: (0, 0)),
        ],
        out_specs=pl.BlockSpec((bn, out_d), lambda i: (i, 0)),
        out_shape=jax.ShapeDtypeStruct((n, out_d), jnp.float32),
        interpret=interpret,
    )(x, agg, deg, W1, b1.reshape(1, h), W2, b2.reshape(1, out_d))

    edge_logits = pl.kernel(
        functools.partial(_sc_edge_body, cpw, k, out_d),
        out_type=jax.ShapeDtypeStruct((e, out_d), jnp.float32),
        mesh=mesh,
        scratch_types=[
            pltpu.VMEM((epw,), jnp.int32),
            pltpu.VMEM((epw,), jnp.int32),
            pltpu.VMEM((k, out_d), jnp.float32),
            pltpu.VMEM((k, out_d), jnp.float32),
            pltpu.VMEM((k, out_d), jnp.float32),
            pltpu.VMEM((k, out_d), jnp.float32),
            pltpu.VMEM((k, out_d), jnp.float32),
            pltpu.VMEM((k, out_d), jnp.float32),
            pltpu.SemaphoreType.DMA,
            pltpu.SemaphoreType.DMA,
            pltpu.SemaphoreType.DMA,
            pltpu.SemaphoreType.DMA,
            pltpu.SemaphoreType.DMA,
            pltpu.SemaphoreType.DMA,
        ],
        interpret=interpret,
    )(nl_half, src_flat, dst_flat)

    return edge_logits


def kernel(x, edge_index, W1, b1, W2, b2):
    return _run(x, edge_index, W1, b1, W2, b2)
